# baseline (speedup 1.0000x reference)
import functools

import jax
import jax.numpy as jnp
from jax.experimental import pallas as pl

NUM_EXPERTS = 64
TOP_K = 8
ROW_BLOCK = 512
NEG = -1e30


def _router_body(x_ref, w_ref, wts_ref, idx_ref, log_ref):
    x = x_ref[...]
    w = w_ref[...]
    logits = jax.lax.dot_general(
        x, w, (((1,), (1,)), ((), ())), preferred_element_type=jnp.float32
    )
    log_ref[...] = logits

    lt = logits.T
    sub = jax.lax.broadcasted_iota(jnp.int32, lt.shape, 0)
    cur = lt
    vals = []
    ids = []
    for _ in range(TOP_K):
        m = jnp.max(cur, axis=0, keepdims=True)
        hit = jnp.where(cur == m, sub, NUM_EXPERTS)
        j = jnp.min(hit, axis=0, keepdims=True)
        vals.append(m)
        ids.append(j)
        cur = jnp.where(sub == j, NEG, cur)
    vt = jnp.concatenate(vals, axis=0)
    e = jnp.exp(vt - vt[0:1])
    wt = e / jnp.sum(e, axis=0, keepdims=True)
    wts_ref[...] = wt.T
    idx_ref[...] = jnp.concatenate(ids, axis=0).T


@functools.partial(jax.jit, static_argnames=())
def kernel(hidden_states, W):
    b, s, h = hidden_states.shape
    n = b * s
    x = hidden_states.reshape(n, h)
    grid = (n // ROW_BLOCK,)
    wts, idx, logits = pl.pallas_call(
        _router_body,
        grid=grid,
        in_specs=[
            pl.BlockSpec((ROW_BLOCK, h), lambda i: (i, 0)),
            pl.BlockSpec((NUM_EXPERTS, h), lambda i: (0, 0)),
        ],
        out_specs=[
            pl.BlockSpec((ROW_BLOCK, TOP_K), lambda i: (i, 0)),
            pl.BlockSpec((ROW_BLOCK, TOP_K), lambda i: (i, 0)),
            pl.BlockSpec((ROW_BLOCK, NUM_EXPERTS), lambda i: (i, 0)),
        ],
        out_shape=[
            jax.ShapeDtypeStruct((n, TOP_K), jnp.float32),
            jax.ShapeDtypeStruct((n, TOP_K), jnp.int32),
            jax.ShapeDtypeStruct((n, NUM_EXPERTS), jnp.float32),
        ],
    )(x, W)
    return (
        wts.reshape(b, s, TOP_K),
        idx.reshape(b, s, TOP_K),
        logits.reshape(b, s, NUM_EXPERTS),
    )


# speedup vs baseline: 1.1755x; 1.1755x over previous
import functools

import jax
import jax.numpy as jnp
from jax.experimental import pallas as pl

NUM_EXPERTS = 64
TOP_K = 8
ROW_BLOCK = 2048
NEG = -1e30


def _router_body(x_ref, w_ref, wts_ref, idx_ref, log_ref):
    x = x_ref[...]
    w = w_ref[...]
    logits = jax.lax.dot_general(
        x, w, (((1,), (1,)), ((), ())), preferred_element_type=jnp.float32
    )
    log_ref[...] = logits

    lt = logits.T
    sub = jax.lax.broadcasted_iota(jnp.int32, lt.shape, 0)
    cur = lt
    vals = []
    ids = []
    for _ in range(TOP_K):
        m = jnp.max(cur, axis=0, keepdims=True)
        hit = jnp.where(cur == m, sub, NUM_EXPERTS)
        j = jnp.min(hit, axis=0, keepdims=True)
        vals.append(m)
        ids.append(j)
        cur = jnp.where(sub == j, NEG, cur)
    vt = jnp.concatenate(vals, axis=0)
    e = jnp.exp(vt - vt[0:1])
    wt = e / jnp.sum(e, axis=0, keepdims=True)
    wts_ref[...] = wt.T
    idx_ref[...] = jnp.concatenate(ids, axis=0).T


@functools.partial(jax.jit, static_argnames=())
def kernel(hidden_states, W):
    b, s, h = hidden_states.shape
    n = b * s
    x = hidden_states.reshape(n, h)
    grid = (n // ROW_BLOCK,)
    wts, idx, logits = pl.pallas_call(
        _router_body,
        grid=grid,
        in_specs=[
            pl.BlockSpec((ROW_BLOCK, h), lambda i: (i, 0)),
            pl.BlockSpec((NUM_EXPERTS, h), lambda i: (0, 0)),
        ],
        out_specs=[
            pl.BlockSpec((ROW_BLOCK, TOP_K), lambda i: (i, 0)),
            pl.BlockSpec((ROW_BLOCK, TOP_K), lambda i: (i, 0)),
            pl.BlockSpec((ROW_BLOCK, NUM_EXPERTS), lambda i: (i, 0)),
        ],
        out_shape=[
            jax.ShapeDtypeStruct((n, TOP_K), jnp.float32),
            jax.ShapeDtypeStruct((n, TOP_K), jnp.int32),
            jax.ShapeDtypeStruct((n, NUM_EXPERTS), jnp.float32),
        ],
    )(x, W)
    return (
        wts.reshape(b, s, TOP_K),
        idx.reshape(b, s, TOP_K),
        logits.reshape(b, s, NUM_EXPERTS),
    )


# speedup vs baseline: 1.1768x; 1.0011x over previous
import jax
import jax.numpy as jnp
from jax.experimental import pallas as pl

NUM_EXPERTS = 64
TOP_K = 8
ROW_BLOCK = 2048
NEG = -1e30


def _router_body(x_ref, w_ref, wts_ref, idx_ref, log_ref):
    x = x_ref[...]
    w = w_ref[...]
    logits = jax.lax.dot_general(
        x, w, (((1,), (1,)), ((), ())), preferred_element_type=jnp.float32
    )
    log_ref[...] = logits

    lt = logits.T
    sub = jax.lax.broadcasted_iota(jnp.int32, lt.shape, 0)
    cur = lt
    vals = []
    ids = []
    for _ in range(TOP_K):
        m = jnp.max(cur, axis=0, keepdims=True)
        hit = jnp.where(cur == m, sub, NUM_EXPERTS)
        j = jnp.min(hit, axis=0, keepdims=True)
        vals.append(m)
        ids.append(j)
        cur = jnp.where(sub == j, NEG, cur)
    vt = jnp.concatenate(vals, axis=0)
    e = jnp.exp(vt - vt[0:1])
    wt = e / jnp.sum(e, axis=0, keepdims=True)
    wts_ref[...] = wt.T
    idx_ref[...] = jnp.concatenate(ids, axis=0).T


def kernel(hidden_states, W):
    b, s, h = hidden_states.shape
    n = b * s
    x = hidden_states.reshape(n, h)
    grid = (n // ROW_BLOCK,)
    wts, idx, logits = pl.pallas_call(
        _router_body,
        grid=grid,
        in_specs=[
            pl.BlockSpec((ROW_BLOCK, h), lambda i: (i, 0)),
            pl.BlockSpec((NUM_EXPERTS, h), lambda i: (0, 0)),
        ],
        out_specs=[
            pl.BlockSpec((ROW_BLOCK, TOP_K), lambda i: (i, 0)),
            pl.BlockSpec((ROW_BLOCK, TOP_K), lambda i: (i, 0)),
            pl.BlockSpec((ROW_BLOCK, NUM_EXPERTS), lambda i: (i, 0)),
        ],
        out_shape=[
            jax.ShapeDtypeStruct((n, TOP_K), jnp.float32),
            jax.ShapeDtypeStruct((n, TOP_K), jnp.int32),
            jax.ShapeDtypeStruct((n, NUM_EXPERTS), jnp.float32),
        ],
    )(x, W)
    return (
        wts.reshape(b, s, TOP_K),
        idx.reshape(b, s, TOP_K),
        logits.reshape(b, s, NUM_EXPERTS),
    )
